# Initial kernel scaffold; baseline (speedup 1.0000x reference)
import jax
import jax.numpy as jnp
from jax.experimental import pallas as pl


def kernel(inputs, degAnchor):
    raise NotImplementedError("write your pallas kernel here")


# speedup vs baseline: 2.0033x; 2.0033x over previous
import jax
import jax.numpy as jnp
from jax import lax
from jax.experimental import pallas as pl

_SCALE = 0.5
_ROWS_PER_BLOCK = 512


def _rot_anchor_block(in_ref, anchor_ref, out_ref):
    x = in_ref[...]
    r, w = x.shape
    depth = w // 2
    cols = lax.broadcasted_iota(jnp.int32, (r, w), 1)
    is_logit = cols < depth
    lx = jnp.where(is_logit, x, -jnp.inf)
    m = jnp.max(lx, axis=1, keepdims=True)
    idx = jnp.min(jnp.where(lx == m, cols, w), axis=1, keepdims=True)
    shift = jnp.sum(jnp.where(cols == idx + depth, x, 0.0), axis=1,
                    keepdims=True)
    anchor = jnp.sum(jnp.where(cols == idx, anchor_ref[...], 0.0), axis=1,
                     keepdims=True)
    out_ref[...] = anchor + shift * _SCALE


def kernel(inputs, degAnchor):
    b, w = inputs.shape
    depth = degAnchor.shape[0]
    r = _ROWS_PER_BLOCK
    anchor_row = jnp.zeros((1, w), jnp.float32).at[0, :depth].set(degAnchor)
    out = pl.pallas_call(
        _rot_anchor_block,
        grid=(b // r,),
        in_specs=[
            pl.BlockSpec((r, w), lambda i: (i, 0)),
            pl.BlockSpec((1, w), lambda i: (0, 0)),
        ],
        out_specs=pl.BlockSpec((r, 1), lambda i: (i, 0)),
        out_shape=jax.ShapeDtypeStruct((b, 1), jnp.float32),
    )(inputs, anchor_row)
    return out[:, 0]
